# baseline (speedup 1.0000x reference)
import functools

import jax
import jax.numpy as jnp
from jax import lax
from jax.experimental import pallas as pl
from jax.experimental.pallas import tpu as pltpu
from jax.experimental.pallas import tpu_sc as plsc

NC = 2
NS = 16
NW = NC * NS
LANES = 16
UNROLL = 8

S_SC = 512
BS = 1792


def _sc_part(x, pos_emb):
    B, S_full, D = x.shape
    base = S_full - S_SC
    RW = S_SC // NW
    CW = RW * D
    NSTEP = B

    xf = x.reshape(B * S_full * D)
    pf = pos_emb.reshape(-1)

    mesh = plsc.VectorSubcoreMesh(core_axis_name="c", subcore_axis_name="s")

    @functools.partial(
        pl.kernel,
        out_type=jax.ShapeDtypeStruct((B * S_SC * D,), jnp.float32),
        mesh=mesh,
        scratch_types=(
            [pltpu.VMEM((CW,), jnp.float32) for _ in range(3)]
            + [pltpu.SemaphoreType.DMA for _ in range(5)]
        ),
    )
    def run(x_hbm, pos_hbm, out_hbm,
            xb0, xb1, pb,
            si0, si1, so0, so1, sp):
        xbufs = [xb0, xb1]
        sin = [si0, si1]
        sout = [so0, so1]

        c = lax.axis_index("c")
        s = lax.axis_index("s")
        wid = s * NC + c
        seq0 = base + wid * RW

        def xoff(b):
            return (b * S_full + seq0) * D

        def ooff(b):
            return (b * S_SC + wid * RW) * D

        def start_xload(b):
            return pltpu.async_copy(
                x_hbm.at[pl.ds(xoff(b), CW)], xbufs[b % 2], sin[b % 2])

        pload = pltpu.async_copy(pos_hbm.at[pl.ds(seq0 * D, CW)], pb, sp)
        xloads = {0: start_xload(0), 1: start_xload(1)}
        stores = {}
        pload.wait()

        for b in range(NSTEP):
            xb = xbufs[b % 2]
            xloads[b].wait()

            def vbody(j, carry):
                for u in range(UNROLL):
                    sl = pl.ds((j * UNROLL + u) * LANES, LANES)
                    plsc.addupdate(xb.at[sl], pb[sl])
                return carry

            lax.fori_loop(0, CW // (LANES * UNROLL), vbody, 0)

            stores[b] = pltpu.async_copy(
                xb, out_hbm.at[pl.ds(ooff(b), CW)], sout[b % 2])

            if b + 2 < NSTEP:
                stores.pop(b).wait()
                xloads[b + 2] = start_xload(b + 2)

        for st in stores.values():
            st.wait()

    return run(xf, pf).reshape(B, S_SC, D)


def _tc_add_kernel(x_ref, p_ref, o_ref):
    o_ref[...] = x_ref[...] + p_ref[...]


def _tc_part(x, pos_emb):
    B, S, D = x.shape
    nblk = (S - S_SC) // BS

    return pl.pallas_call(
        _tc_add_kernel,
        grid=(nblk, B),
        in_specs=[
            pl.BlockSpec((1, BS, D), lambda i, b: (b, i, 0)),
            pl.BlockSpec((BS, D), lambda i, b: (i, 0)),
        ],
        out_specs=pl.BlockSpec((1, BS, D), lambda i, b: (b, i, 0)),
        out_shape=jax.ShapeDtypeStruct((B, S, D), jnp.float32),
    )(x, pos_emb)


def kernel(x, pos_emb):
    B, S, D = x.shape
    sc = _sc_part(x, pos_emb)
    big = _tc_part(x, pos_emb)
    return lax.dynamic_update_slice(big, sc, (0, S - S_SC, 0))


# speedup vs baseline: 3.1418x; 3.1418x over previous
import jax
import jax.numpy as jnp
from jax.experimental import pallas as pl

BS = 2048


def _add_kernel(x_ref, p_ref, o_ref):
    o_ref[...] = x_ref[...] + p_ref[...]


def kernel(x, pos_emb):
    B, S, D = x.shape
    return pl.pallas_call(
        _add_kernel,
        grid=(S // BS, B),
        in_specs=[
            pl.BlockSpec((1, BS, D), lambda i, b: (b, i, 0)),
            pl.BlockSpec((BS, D), lambda i, b: (i, 0)),
        ],
        out_specs=pl.BlockSpec((1, BS, D), lambda i, b: (b, i, 0)),
        out_shape=jax.ShapeDtypeStruct((B, S, D), jnp.float32),
    )(x, pos_emb)
